# baseline (speedup 1.0000x reference)
import functools

import jax
import jax.numpy as jnp
from jax import lax
from jax.experimental import pallas as pl
from jax.experimental.pallas import tpu as pltpu
from jax.experimental.pallas import tpu_sc as plsc

_NF = 26
_V = 100001
_D = 32
_B = 1024
_L = 50
_NTOK = _B * _L
_NCORES = 2
_NSUB = 16
_NW = _NCORES * _NSUB
_T = 128
_NCHUNK = _NTOK // _T
_NFLD = _NF + 1
_TU = (_NF * _V * _D + 127) // 128
_PU = (_V * _D + 127) // 128


def _make_kernel():
    mesh = plsc.VectorSubcoreMesh(core_axis_name="c", subcore_axis_name="s")

    @functools.partial(
        pl.kernel,
        out_type=jax.ShapeDtypeStruct((_NTOK * _NFLD, 1, 128), jnp.float32),
        mesh=mesh,
        scratch_types=[
            pltpu.VMEM((_NFLD, _T), jnp.int32),
            pltpu.VMEM((_NFLD, _T), jnp.int32),
            pltpu.VMEM((_T, 1, 128), jnp.float32),
            pltpu.VMEM((_T, 1, 128), jnp.float32),
            pltpu.SemaphoreType.DMA,
            pltpu.SemaphoreType.DMA,
        ],
    )
    def emb(goff, soff, tables, ptable, out, goff_v, soff_v, rows0, rows1,
            gsem, ssem):
        w = lax.axis_index("c") * _NSUB + lax.axis_index("s")
        nchunks = jnp.where(w < _NCHUNK - 12 * _NW, 13, 12)
        bufs = (rows0, rows1)

        def body(i, carry):
            c = w + i * _NW
            pltpu.sync_copy(goff.at[c], goff_v)
            pltpu.sync_copy(soff.at[c], soff_v)
            gh = [None] * _NFLD
            sh = [None] * _NFLD
            for f in range(_NFLD):
                buf = bufs[f % 2]
                if f >= 2:
                    sh[f - 2].wait()
                src = tables if f < _NF else ptable
                gh[f] = pltpu.async_copy(src.at[goff_v.at[f]], buf, gsem)
                if f >= 1:
                    gh[f - 1].wait()
                    sh[f - 1] = pltpu.async_copy(
                        bufs[(f - 1) % 2], out.at[soff_v.at[f - 1]], ssem
                    )
            gh[_NFLD - 1].wait()
            sh[_NFLD - 1] = pltpu.async_copy(
                bufs[(_NFLD - 1) % 2], out.at[soff_v.at[_NFLD - 1]], ssem
            )
            sh[_NFLD - 2].wait()
            sh[_NFLD - 1].wait()
            return carry

        lax.fori_loop(0, nchunks, body, 0)

    return emb


_EMB = _make_kernel()


def kernel(transactions_cat_features, product_feature, tables, product_table):
    trans = transactions_cat_features.astype(jnp.int32)
    foff = (jnp.arange(_NF, dtype=jnp.int32) * _V)[:, None, None]
    r_fields = (trans.reshape(_NF, _NCHUNK, _T) + foff).transpose(1, 0, 2)
    r_prod = jnp.broadcast_to(
        product_feature.astype(jnp.int32)[:, None], (_B, _L)
    ).reshape(_NCHUNK, 1, _T)
    rall = jnp.concatenate([r_fields, r_prod], axis=1)
    goff = lax.shift_right_logical(rall, 2)
    sub = jnp.bitwise_and(rall, 3)
    tok = jnp.arange(_NTOK, dtype=jnp.int32).reshape(_NCHUNK, 1, _T)
    fld = jnp.arange(_NFLD, dtype=jnp.int32).reshape(1, _NFLD, 1)
    soff = tok * _NFLD + fld
    tflat = jnp.pad(tables.reshape(-1), (0, _TU * 128 - _NF * _V * _D))
    tables_d = tflat.reshape(_TU, 1, 128)
    pflat = jnp.pad(product_table.reshape(-1), (0, _PU * 128 - _V * _D))
    ptable_d = pflat.reshape(_PU, 1, 128)
    out_u = _EMB(goff, soff, tables_d, ptable_d)
    units = out_u.reshape(_NCHUNK, _T, _NFLD, 4, _D)
    sub_t = sub.transpose(0, 2, 1)
    onehot = (
        sub_t[..., None] == jnp.arange(4, dtype=jnp.int32)
    ).astype(jnp.float32)
    picked = jnp.einsum("ctfsd,ctfs->ctfd", units, onehot)
    return picked.reshape(_B, _L, _NFLD * _D)


# speedup vs baseline: 1.4356x; 1.4356x over previous
import functools

import jax
import jax.numpy as jnp
from jax import lax
from jax.experimental import pallas as pl
from jax.experimental.pallas import tpu as pltpu
from jax.experimental.pallas import tpu_sc as plsc

_NF = 26
_V = 100001
_D = 32
_B = 1024
_L = 50
_NTOK = _B * _L
_NCORES = 2
_NSUB = 16
_NW = _NCORES * _NSUB
_T = 128
_NCHUNK = _NTOK // _T
_NFLD = _NF + 1
_TU = (_NF * _V * _D + 127) // 128
_PU = (_V * _D + 127) // 128


def _make_kernel():
    mesh = plsc.VectorSubcoreMesh(core_axis_name="c", subcore_axis_name="s")

    @functools.partial(
        pl.kernel,
        out_type=jax.ShapeDtypeStruct((_NTOK * _NFLD, 1, 128), jnp.float32),
        mesh=mesh,
        scratch_types=[
            pltpu.VMEM((_NFLD, _T), jnp.int32),
            pltpu.VMEM((_NFLD, _T), jnp.int32),
            pltpu.VMEM((_T, 1, 128), jnp.float32),
            pltpu.VMEM((_T, 1, 128), jnp.float32),
            pltpu.SemaphoreType.DMA,
            pltpu.SemaphoreType.DMA,
        ],
    )
    def emb(goff, soff, tables, ptable, out, goff_v, soff_v, rows0, rows1,
            gsem, ssem):
        w = lax.axis_index("c") * _NSUB + lax.axis_index("s")
        nchunks = jnp.where(w < _NCHUNK - 12 * _NW, 13, 12)
        bufs = (rows0, rows1)

        def body(i, carry):
            c = w + i * _NW
            pltpu.sync_copy(goff.at[c], goff_v)
            pltpu.sync_copy(soff.at[c], soff_v)
            gh = [None] * _NFLD
            sh = [None] * _NFLD
            for f in range(_NFLD):
                buf = bufs[f % 2]
                if f >= 2:
                    sh[f - 2].wait()
                src = tables if f < _NF else ptable
                gh[f] = pltpu.async_copy(src.at[goff_v.at[f]], buf, gsem)
                if f >= 1:
                    gh[f - 1].wait()
                    sh[f - 1] = pltpu.async_copy(
                        bufs[(f - 1) % 2], out.at[soff_v.at[f - 1]], ssem
                    )
            gh[_NFLD - 1].wait()
            sh[_NFLD - 1] = pltpu.async_copy(
                bufs[(_NFLD - 1) % 2], out.at[soff_v.at[_NFLD - 1]], ssem
            )
            sh[_NFLD - 2].wait()
            sh[_NFLD - 1].wait()
            return carry

        lax.fori_loop(0, nchunks, body, 0)

    return emb


_EMB = _make_kernel()


def kernel(transactions_cat_features, product_feature, tables, product_table):
    trans = transactions_cat_features.astype(jnp.int32)
    foff = (jnp.arange(_NF, dtype=jnp.int32) * _V)[:, None, None]
    r_fields = (trans.reshape(_NF, _NCHUNK, _T) + foff).transpose(1, 0, 2)
    r_prod = jnp.broadcast_to(
        product_feature.astype(jnp.int32)[:, None], (_B, _L)
    ).reshape(_NCHUNK, 1, _T)
    rall = jnp.concatenate([r_fields, r_prod], axis=1)
    goff = lax.shift_right_logical(rall, 2)
    sub = jnp.bitwise_and(rall, 3)
    tok = jnp.arange(_NTOK, dtype=jnp.int32).reshape(_NCHUNK, 1, _T)
    fld = jnp.arange(_NFLD, dtype=jnp.int32).reshape(1, _NFLD, 1)
    soff = tok * _NFLD + fld
    one = (product_feature[0] * 0 + 1).astype(jnp.float32)
    tflat = jnp.pad(
        (tables * one).reshape(-1), (0, _TU * 128 - _NF * _V * _D)
    )
    tables_d = tflat.reshape(_TU, 1, 128)
    pflat = jnp.pad(
        (product_table * one).reshape(-1), (0, _PU * 128 - _V * _D)
    )
    ptable_d = pflat.reshape(_PU, 1, 128)
    out_u = _EMB(goff, soff, tables_d, ptable_d)
    out2d = out_u.reshape(_NTOK * _NFLD, 128)
    sub_flat = sub.transpose(0, 2, 1).reshape(_NTOK * _NFLD)
    lane_s = (jnp.arange(128, dtype=jnp.int32) >> 5)[None, :]
    masked = out2d * (lane_s == sub_flat[:, None]).astype(jnp.float32)
    picked = masked.reshape(_NTOK * _NFLD, 4, _D).sum(axis=1)
    return picked.reshape(_B, _L, _NFLD * _D)


# speedup vs baseline: 1.4784x; 1.0299x over previous
import functools

import jax
import jax.numpy as jnp
from jax import lax
from jax.experimental import pallas as pl
from jax.experimental.pallas import tpu as pltpu
from jax.experimental.pallas import tpu_sc as plsc

_NF = 26
_V = 100001
_D = 32
_B = 1024
_L = 50
_NTOK = _B * _L
_NCORES = 2
_NSUB = 16
_NW = _NCORES * _NSUB
_T = 128
_NCHUNK = _NTOK // _T
_NFLD = _NF + 1
_TU = (_NF * _V * _D + 127) // 128
_PU = (_V * _D + 127) // 128


def _make_kernel():
    mesh = plsc.VectorSubcoreMesh(core_axis_name="c", subcore_axis_name="s")

    @functools.partial(
        pl.kernel,
        out_type=jax.ShapeDtypeStruct((_NTOK * _NFLD, 1, 128), jnp.float32),
        mesh=mesh,
        scratch_types=[
            pltpu.VMEM((_NFLD, _T), jnp.int32),
            pltpu.VMEM((_NFLD, _T), jnp.int32),
            pltpu.VMEM((_T, 1, 128), jnp.float32),
            pltpu.VMEM((_T, 1, 128), jnp.float32),
            pltpu.SemaphoreType.DMA,
            pltpu.SemaphoreType.DMA,
        ],
    )
    def emb(goff, soff, tables, ptable, out, goff_v, soff_v, rows0, rows1,
            gsem, ssem):
        w = lax.axis_index("c") * _NSUB + lax.axis_index("s")
        nchunks = jnp.where(w < _NCHUNK - 12 * _NW, 13, 12)
        bufs = (rows0, rows1)

        def body(i, carry):
            c = w + i * _NW
            pltpu.sync_copy(goff.at[c], goff_v)
            pltpu.sync_copy(soff.at[c], soff_v)
            gh = [None] * _NFLD
            sh = [None] * _NFLD
            for f in range(_NFLD):
                buf = bufs[f % 2]
                if f >= 2:
                    sh[f - 2].wait()
                src = tables if f < _NF else ptable
                gh[f] = pltpu.async_copy(src.at[goff_v.at[f]], buf, gsem)
                if f >= 1:
                    gh[f - 1].wait()
                    sh[f - 1] = pltpu.async_copy(
                        bufs[(f - 1) % 2], out.at[soff_v.at[f - 1]], ssem
                    )
            gh[_NFLD - 1].wait()
            sh[_NFLD - 1] = pltpu.async_copy(
                bufs[(_NFLD - 1) % 2], out.at[soff_v.at[_NFLD - 1]], ssem
            )
            sh[_NFLD - 2].wait()
            sh[_NFLD - 1].wait()
            return carry

        lax.fori_loop(0, nchunks, body, 0)

    return emb


_EMB = _make_kernel()


def kernel(transactions_cat_features, product_feature, tables, product_table):
    trans = transactions_cat_features.astype(jnp.int32)
    foff = (jnp.arange(_NF, dtype=jnp.int32) * _V)[:, None, None]
    r_fields = (trans.reshape(_NF, _NCHUNK, _T) + foff).transpose(1, 0, 2)
    r_prod = jnp.broadcast_to(
        product_feature.astype(jnp.int32)[:, None], (_B, _L)
    ).reshape(_NCHUNK, 1, _T)
    rall = jnp.concatenate([r_fields, r_prod], axis=1)
    goff = lax.shift_right_logical(rall, 2)
    sub = jnp.bitwise_and(rall, 3)
    tok = jnp.arange(_NTOK, dtype=jnp.int32).reshape(_NCHUNK, 1, _T)
    fld = jnp.arange(_NFLD, dtype=jnp.int32).reshape(1, _NFLD, 1)
    soff = tok * _NFLD + fld
    one = (product_feature[0] * 0 + 1).astype(jnp.float32)
    tflat = jnp.pad(
        (tables * one).reshape(-1), (0, _TU * 128 - _NF * _V * _D)
    )
    tables_d = tflat.reshape(_TU, 1, 128)
    pflat = jnp.pad(
        (product_table * one).reshape(-1), (0, _PU * 128 - _V * _D)
    )
    ptable_d = pflat.reshape(_PU, 1, 128)
    out_u = _EMB(goff, soff, tables_d, ptable_d)
    out2d = out_u.reshape(_NTOK * _NFLD, 128)
    sub_flat = sub.transpose(0, 2, 1).reshape(_NTOK * _NFLD)
    lane_s = (jnp.arange(128, dtype=jnp.int32) >> 5)[None, :]
    masked = out2d * (lane_s == sub_flat[:, None]).astype(jnp.float32)
    picked = (
        masked[:, 0:32] + masked[:, 32:64] + masked[:, 64:96] + masked[:, 96:128]
    )
    return picked.reshape(_B, _L, _NFLD * _D)


# speedup vs baseline: 5.1423x; 3.4782x over previous
import functools

import jax
import jax.numpy as jnp
from jax import lax
from jax.experimental import pallas as pl
from jax.experimental.pallas import tpu as pltpu
from jax.experimental.pallas import tpu_sc as plsc

_NF = 26
_V = 100001
_VP = 100096
_D = 32
_B = 1024
_L = 50
_NTOK = _B * _L
_NCORES = 2
_NSUB = 16
_NW = _NCORES * _NSUB
_NFLD = _NF + 1
_NU = _NFLD * _D
_UPW = _NU // _NW
_C = 6400
_NC = _NTOK // _C


def _make_kernel():
    mesh = plsc.VectorSubcoreMesh(core_axis_name="c", subcore_axis_name="s")

    @functools.partial(
        pl.kernel,
        out_type=jax.ShapeDtypeStruct((_NU, 1, _NTOK), jnp.float32),
        mesh=mesh,
        compiler_params=pltpu.CompilerParams(needs_layout_passes=False),
        scratch_types=[
            pltpu.VMEM((_VP,), jnp.float32),
            pltpu.VMEM((1, _C), jnp.int32),
            pltpu.VMEM((1, _C), jnp.float32),
        ],
    )
    def emb(flat, idx_all, out, slab_v, idx_v, o_v):
        w = lax.axis_index("c") * _NSUB + lax.axis_index("s")
        iota16 = lax.iota(jnp.int32, 16)
        zero16 = iota16 * 0

        def unit_body(j, carry):
            u = w * _UPW + j
            f = lax.shift_right_logical(u, 5)
            pltpu.sync_copy(flat.at[pl.ds(u * _VP, _VP)], slab_v)
            for c in range(_NC):
                pltpu.sync_copy(
                    idx_all.at[f, :, pl.ds(c * _C, _C)], idx_v
                )

                def blk(b, carry2):
                    for k in range(8):
                        lane = b * 128 + k * 16 + iota16
                        iv = plsc.load_gather(idx_v, [zero16, lane])
                        vals = plsc.load_gather(slab_v, [iv])
                        plsc.store_scatter(o_v, [zero16, lane], vals)
                    return carry2

                lax.fori_loop(0, _C // 128, blk, 0)
                pltpu.sync_copy(o_v, out.at[u, :, pl.ds(c * _C, _C)])
            return carry

        lax.fori_loop(0, _UPW, unit_body, 0)

    return emb


_EMB = _make_kernel()


def kernel(transactions_cat_features, product_feature, tables, product_table):
    trans = transactions_cat_features.astype(jnp.int32)
    idx_f = trans.reshape(_NF, _NTOK)
    idx_p = jnp.broadcast_to(
        product_feature.astype(jnp.int32)[:, None], (_B, _L)
    ).reshape(1, _NTOK)
    idx_all = jnp.concatenate([idx_f, idx_p], axis=0).reshape(_NFLD, 1, _NTOK)
    tpad = jnp.pad(
        jnp.transpose(tables, (0, 2, 1)), ((0, 0), (0, 0), (0, _VP - _V))
    ).reshape(_NF * _D * _VP)
    ppad = jnp.pad(
        jnp.transpose(product_table, (1, 0)), ((0, 0), (0, _VP - _V))
    ).reshape(_D * _VP)
    flat = jnp.concatenate([tpad, ppad])
    out_t = _EMB(flat, idx_all)
    return out_t.reshape(_NU, _NTOK).T.reshape(_B, _L, _NU)


# speedup vs baseline: 5.6566x; 1.1000x over previous
import functools

import jax
import jax.numpy as jnp
from jax import lax
from jax.experimental import pallas as pl
from jax.experimental.pallas import tpu as pltpu
from jax.experimental.pallas import tpu_sc as plsc

_NF = 26
_V = 100001
_VP = 100096
_D = 32
_B = 1024
_L = 50
_NTOK = _B * _L
_NCORES = 2
_NSUB = 16
_NW = _NCORES * _NSUB
_NFLD = _NF + 1
_NU = _NFLD * _D
_UPW = _NU // _NW
_C = 6400
_NC = _NTOK // _C


def _make_kernel():
    mesh = plsc.VectorSubcoreMesh(core_axis_name="c", subcore_axis_name="s")

    @functools.partial(
        pl.kernel,
        out_type=jax.ShapeDtypeStruct((_NU, 1, _NTOK), jnp.float32),
        mesh=mesh,
        compiler_params=pltpu.CompilerParams(needs_layout_passes=False),
        scratch_types=[
            pltpu.VMEM((_VP,), jnp.float32),
            pltpu.VMEM((1, _C), jnp.int32),
            pltpu.VMEM((1, _C), jnp.int32),
            pltpu.VMEM((1, _C), jnp.float32),
            pltpu.VMEM((1, _C), jnp.float32),
            pltpu.SemaphoreType.DMA,
            pltpu.SemaphoreType.DMA,
        ],
    )
    def emb(flat, idx_all, out, slab_v, idx0, idx1, ov0, ov1, isem, osem):
        w = lax.axis_index("c") * _NSUB + lax.axis_index("s")
        iota16 = lax.iota(jnp.int32, 16)
        zero16 = iota16 * 0
        ibufs = (idx0, idx1)
        obufs = (ov0, ov1)

        def unit_body(j, carry):
            u = w * _UPW + j
            f = lax.shift_right_logical(u, 5)
            pltpu.sync_copy(flat.at[pl.ds(u * _VP, _VP)], slab_v)
            ih = [
                pltpu.async_copy(
                    idx_all.at[f, :, pl.ds(0, _C)], ibufs[0], isem
                )
            ]
            oh = []
            for c in range(_NC):
                bi = c % 2
                ih[c].wait()
                if c + 1 < _NC:
                    ih.append(
                        pltpu.async_copy(
                            idx_all.at[f, :, pl.ds((c + 1) * _C, _C)],
                            ibufs[1 - bi], isem,
                        )
                    )
                if c >= 2:
                    oh[c - 2].wait()
                idx_v = ibufs[bi]
                o_v = obufs[bi]

                def blk(b, carry2):
                    for k in range(8):
                        lane = b * 128 + k * 16 + iota16
                        iv = plsc.load_gather(idx_v, [zero16, lane])
                        vals = plsc.load_gather(slab_v, [iv])
                        plsc.store_scatter(o_v, [zero16, lane], vals)
                    return carry2

                lax.fori_loop(0, _C // 128, blk, 0)
                oh.append(
                    pltpu.async_copy(
                        o_v, out.at[u, :, pl.ds(c * _C, _C)], osem
                    )
                )
            oh[_NC - 2].wait()
            oh[_NC - 1].wait()
            return carry

        lax.fori_loop(0, _UPW, unit_body, 0)

    return emb


_EMB = _make_kernel()


def kernel(transactions_cat_features, product_feature, tables, product_table):
    trans = transactions_cat_features.astype(jnp.int32)
    idx_f = trans.reshape(_NF, _NTOK)
    idx_p = jnp.broadcast_to(
        product_feature.astype(jnp.int32)[:, None], (_B, _L)
    ).reshape(1, _NTOK)
    idx_all = jnp.concatenate([idx_f, idx_p], axis=0).reshape(_NFLD, 1, _NTOK)
    tpad = jnp.pad(
        jnp.transpose(tables, (0, 2, 1)), ((0, 0), (0, 0), (0, _VP - _V))
    ).reshape(_NF * _D * _VP)
    ppad = jnp.pad(
        jnp.transpose(product_table, (1, 0)), ((0, 0), (0, _VP - _V))
    ).reshape(_D * _VP)
    flat = jnp.concatenate([tpad, ppad])
    out_t = _EMB(flat, idx_all)
    return out_t.reshape(_NU, _NTOK).T.reshape(_B, _L, _NU)


# speedup vs baseline: 5.8763x; 1.0388x over previous
import functools

import jax
import jax.numpy as jnp
from jax import lax
from jax.experimental import pallas as pl
from jax.experimental.pallas import tpu as pltpu
from jax.experimental.pallas import tpu_sc as plsc

_NF = 26
_V = 100001
_VP = 100096
_D = 32
_B = 1024
_L = 50
_NTOK = _B * _L
_NCORES = 2
_NSUB = 16
_NW = _NCORES * _NSUB
_NFLD = _NF + 1
_NU = _NFLD * _D
_UPW = _NU // _NW
_C = 6400
_NC = _NTOK // _C


def _make_kernel():
    mesh = plsc.VectorSubcoreMesh(core_axis_name="c", subcore_axis_name="s")

    @functools.partial(
        pl.kernel,
        out_type=jax.ShapeDtypeStruct((_NU, 1, _NTOK), jnp.float32),
        mesh=mesh,
        compiler_params=pltpu.CompilerParams(needs_layout_passes=False),
        scratch_types=[
            pltpu.VMEM((_VP,), jnp.float32),
            pltpu.VMEM((1, _C), jnp.int32),
            pltpu.VMEM((1, _C), jnp.int32),
            pltpu.VMEM((1, _C), jnp.float32),
            pltpu.VMEM((1, _C), jnp.float32),
            pltpu.SemaphoreType.DMA,
            pltpu.SemaphoreType.DMA,
        ],
    )
    def emb(flat, idx_all, out, slab_v, idx0, idx1, ov0, ov1, isem, osem):
        w = lax.axis_index("c") * _NSUB + lax.axis_index("s")
        iota16 = lax.iota(jnp.int32, 16)
        zero16 = iota16 * 0
        ibufs = (idx0, idx1)
        obufs = (ov0, ov1)

        def unit_body(j, carry):
            u = w * _UPW + j
            f = lax.shift_right_logical(u, 5)
            pltpu.sync_copy(flat.at[pl.ds(u * _VP, _VP)], slab_v)
            ih = [
                pltpu.async_copy(
                    idx_all.at[f, :, pl.ds(0, _C)], ibufs[0], isem
                )
            ]
            oh = []
            for c in range(_NC):
                bi = c % 2
                ih[c].wait()
                if c + 1 < _NC:
                    ih.append(
                        pltpu.async_copy(
                            idx_all.at[f, :, pl.ds((c + 1) * _C, _C)],
                            ibufs[1 - bi], isem,
                        )
                    )
                if c >= 2:
                    oh[c - 2].wait()
                idx_v = ibufs[bi]
                o_v = obufs[bi]

                def blk(b, carry2):
                    for k in range(8):
                        lane = b * 128 + k * 16 + iota16
                        iv = plsc.load_gather(idx_v, [zero16, lane])
                        vals = plsc.load_gather(slab_v, [iv])
                        plsc.store_scatter(o_v, [zero16, lane], vals)
                    return carry2

                lax.fori_loop(0, _C // 128, blk, 0)
                oh.append(
                    pltpu.async_copy(
                        o_v, out.at[u, :, pl.ds(c * _C, _C)], osem
                    )
                )
            oh[_NC - 2].wait()
            oh[_NC - 1].wait()
            return carry

        lax.fori_loop(0, _UPW, unit_body, 0)

    return emb


_EMB = _make_kernel()


def kernel(transactions_cat_features, product_feature, tables, product_table):
    trans = transactions_cat_features.astype(jnp.int32)
    idx_f = trans.reshape(_NF, _NTOK)
    idx_p = jnp.broadcast_to(
        product_feature.astype(jnp.int32)[:, None], (_B, _L)
    ).reshape(1, _NTOK)
    idx_all = jnp.concatenate([idx_f, idx_p], axis=0).reshape(_NFLD, 1, _NTOK)
    tpad = jnp.pad(
        jnp.transpose(tables, (0, 2, 1)), ((0, 0), (0, 0), (0, _VP - _V))
    ).reshape(_NF * _D * _VP)
    ppad = jnp.pad(
        jnp.transpose(product_table, (1, 0)), ((0, 0), (0, _VP - _V))
    ).reshape(_D * _VP)
    flat = jnp.concatenate([tpad, ppad])
    out_t = _EMB(flat, idx_all)
    eye = jnp.eye(_NU, dtype=jnp.float32)
    picked = jax.lax.dot_general(
        out_t.reshape(_NU, _NTOK), eye, (((0,), (0,)), ((), ())),
        preferred_element_type=jnp.float32,
    )
    return picked.reshape(_B, _L, _NU)
